# baseline (speedup 1.0000x reference)
import functools

import jax
import jax.numpy as jnp
from jax import lax
from jax.experimental import pallas as pl
from jax.experimental.pallas import tpu as pltpu
from jax.experimental.pallas import tpu_sc as plsc

_VOCAB = 4096
_N = 8 * 2048
_NC, _NS, _L = 2, 16, 16
_NW = _NC * _NS
_ROWS_PER_W = _N // _NW
_K = 8
_NB = 2
_NCHUNK = _ROWS_PER_W // _K
_NG = _NCHUNK // _NB
_TPAD = _ROWS_PER_W + _L


def _lse_body(w_ref, out_ref):
    x = w_ref[...]
    m = jnp.max(x, axis=1, keepdims=True)
    s = jnp.sum(jnp.exp(x - m), axis=1, keepdims=True)
    out_ref[...] = m + jnp.log(s)


_R = 256


def _compute_lse(W):
    return pl.pallas_call(
        _lse_body,
        grid=(_VOCAB // _R,),
        in_specs=[pl.BlockSpec((_R, _VOCAB), lambda i: (i, 0))],
        out_specs=pl.BlockSpec((_R, 1), lambda i: (i, 0)),
        out_shape=jax.ShapeDtypeStruct((_VOCAB, 1), jnp.float32),
    )(W)


def _combine_body(hist_ref, lse_ref, tpart_ref, out_ref):
    counts = jnp.sum(hist_ref[...].astype(jnp.float32), axis=0,
                     keepdims=True)
    s_lse = jnp.dot(counts, lse_ref[...],
                    preferred_element_type=jnp.float32)
    s_tv = jnp.sum(tpart_ref[...])
    out_ref[...] = (s_lse - s_tv) / _N


def _combine(hist, lse2, tpart):
    return pl.pallas_call(
        _combine_body,
        out_shape=jax.ShapeDtypeStruct((1, 1), jnp.float32),
    )(hist, lse2, tpart)


@functools.partial(
    pl.kernel,
    mesh=plsc.VectorSubcoreMesh(core_axis_name="c", subcore_axis_name="s"),
    out_type=[
        jax.ShapeDtypeStruct((_N, _VOCAB), jnp.float32),
        jax.ShapeDtypeStruct((_NW, _L), jnp.float32),
        jax.ShapeDtypeStruct((_NC, _VOCAB), jnp.float32),
    ],
    scratch_types=[
        pltpu.VMEM((_ROWS_PER_W,), jnp.int32),
        pltpu.VMEM((_NCHUNK, _K), jnp.int32),
        pltpu.VMEM((_TPAD,), jnp.int32),
        pltpu.VMEM((_ROWS_PER_W,), jnp.float32),
        pltpu.VMEM((_VOCAB // _NS,), jnp.float32),
        pltpu.VMEM((_NB, _K, _VOCAB), jnp.float32),
        pltpu.VMEM((_L,), jnp.float32),
        pltpu.VMEM_SHARED((_VOCAB,), jnp.float32),
        pltpu.SemaphoreType.DMA((_NB,)),
        pltpu.SemaphoreType.DMA((_NB,)),
        pltpu.SemaphoreType.DMA,
    ],
)
def _sc_gather(W_hbm, idx_hbm, idx2_hbm, tgt_hbm, out_hbm, part_hbm, hist_hbm,
               idx_v, idx2_v, tgt_v, ones_v, zsrc_v, rows_v, acc_v, hist_sh,
               gsems, wsems, hsem):
    sid = lax.axis_index("s")
    cid = lax.axis_index("c")
    wid = sid * _NC + cid
    base = wid * _ROWS_PER_W
    pltpu.sync_copy(idx_hbm.at[pl.ds(base, _ROWS_PER_W)], idx_v)
    pltpu.sync_copy(idx2_hbm.at[pl.ds(wid * _NCHUNK, _NCHUNK)], idx2_v)
    pltpu.sync_copy(tgt_hbm.at[pl.ds(base, _ROWS_PER_W)],
                    tgt_v.at[pl.ds(0, _ROWS_PER_W)])

    lanes = lax.iota(jnp.int32, _L)

    def _gather(c, b):
        return pltpu.async_copy(
            W_hbm.at[idx2_v.at[c]], rows_v.at[b], gsems.at[b])

    def _gather_wait(c, b):
        pltpu.make_async_copy(
            W_hbm.at[idx2_v.at[c]], rows_v.at[b],
            gsems.at[b]).wait()

    def _wb(c, b):
        return pltpu.async_copy(
            rows_v.at[b], out_hbm.at[pl.ds(base + c * _K, _K)], wsems.at[b])

    def _wb_wait(c, b):
        pltpu.make_async_copy(
            rows_v.at[b], out_hbm.at[pl.ds(base + c * _K, _K)],
            wsems.at[b]).wait()

    _gather(0, 0)
    _gather(1, 1)

    _ZS = _VOCAB // _NS

    def obody(j, _):
        ones_v[pl.ds(j * _L, _L)] = jnp.ones((_L,), jnp.float32)
        return 0

    lax.fori_loop(0, _ROWS_PER_W // _L, obody, 0)

    def zb(j, _):
        zsrc_v[pl.ds(j * _L, _L)] = jnp.zeros((_L,), jnp.float32)
        return 0

    lax.fori_loop(0, _ZS // _L, zb, 0)
    pltpu.sync_copy(zsrc_v.at[pl.ds(0, _ZS)], hist_sh.at[pl.ds(sid * _ZS, _ZS)])
    plsc.subcore_barrier()
    cp_h = pltpu.async_copy(ones_v, hist_sh.at[idx_v], hsem, add=True)

    def _chunk_tvals(b, t16, accv):
        for j in range(_K):
            t_j = t16[j]
            cbase = (t_j // _L) * _L
            sl = rows_v[b, j, pl.ds(cbase, _L)]
            accv = accv + jnp.where(lanes == (t_j % _L), sl, 0.0)
        return accv

    def body(g, acct):
        e = 2 * g
        o = e + 1
        _gather_wait(e, 0)
        _wb(e, 0)
        acct = _chunk_tvals(0, tgt_v[pl.ds(e * _K, _L)], acct)
        _gather_wait(o, 1)
        _wb(o, 1)
        acct = _chunk_tvals(1, tgt_v[pl.ds(o * _K, _L)], acct)
        _wb_wait(e, 0)

        @pl.when(g < _NG - 1)
        def _():
            _gather(e + 2, 0)

        _wb_wait(o, 1)

        @pl.when(g < _NG - 1)
        def _():
            _gather(o + 2, 1)
        return acct

    acct = lax.fori_loop(0, _NG, body, jnp.zeros((_L,), jnp.float32))

    acc_v[...] = acct
    pltpu.sync_copy(acc_v, part_hbm.at[wid])

    cp_h.wait()
    plsc.subcore_barrier()

    @pl.when(sid == 0)
    def _():
        pltpu.sync_copy(hist_sh, hist_hbm.at[cid])


def kernel(idx, targets, W):
    idx_flat = idx.reshape(_N).astype(jnp.int32)
    tgt_flat = targets.reshape(_N).astype(jnp.int32)
    lse2 = _compute_lse(W)
    logits_flat, tpart, hist = _sc_gather(
        W, idx_flat, idx_flat.reshape(_N // _K, _K), tgt_flat)
    loss = _combine(hist, lse2, tpart)[0, 0]
    return (logits_flat, loss)


# speedup vs baseline: 1.0351x; 1.0351x over previous
import functools

import jax
import jax.numpy as jnp
from jax import lax
from jax.experimental import pallas as pl
from jax.experimental.pallas import tpu as pltpu
from jax.experimental.pallas import tpu_sc as plsc

_VOCAB = 4096
_N = 8 * 2048
_NC, _NS, _L = 2, 16, 16
_NW = _NC * _NS
_ROWS_PER_W = _N // _NW
_K = 4
_NB = 4
_NCHUNK = _ROWS_PER_W // _K
_NG = _NCHUNK // _NB
_TPAD = _ROWS_PER_W + _L


def _lse_body(w_ref, out_ref):
    x = w_ref[...]
    m = jnp.max(x, axis=1, keepdims=True)
    s = jnp.sum(jnp.exp(x - m), axis=1, keepdims=True)
    out_ref[...] = m + jnp.log(s)


_R = 256


def _compute_lse(W):
    return pl.pallas_call(
        _lse_body,
        grid=(_VOCAB // _R,),
        in_specs=[pl.BlockSpec((_R, _VOCAB), lambda i: (i, 0))],
        out_specs=pl.BlockSpec((_R, 1), lambda i: (i, 0)),
        out_shape=jax.ShapeDtypeStruct((_VOCAB, 1), jnp.float32),
    )(W)


def _combine_body(hist_ref, lse_ref, tpart_ref, out_ref):
    counts = jnp.sum(hist_ref[...].astype(jnp.float32), axis=0,
                     keepdims=True)
    s_lse = jnp.dot(counts, lse_ref[...],
                    preferred_element_type=jnp.float32)
    s_tv = jnp.sum(tpart_ref[...])
    out_ref[...] = (s_lse - s_tv) / _N


def _combine(hist, lse2, tpart):
    return pl.pallas_call(
        _combine_body,
        out_shape=jax.ShapeDtypeStruct((1, 1), jnp.float32),
    )(hist, lse2, tpart)


@functools.partial(
    pl.kernel,
    mesh=plsc.VectorSubcoreMesh(core_axis_name="c", subcore_axis_name="s"),
    out_type=[
        jax.ShapeDtypeStruct((_N, _VOCAB), jnp.float32),
        jax.ShapeDtypeStruct((_NW, _L), jnp.float32),
        jax.ShapeDtypeStruct((_NC, _VOCAB), jnp.float32),
    ],
    scratch_types=[
        pltpu.VMEM((_ROWS_PER_W,), jnp.int32),
        pltpu.VMEM((_NCHUNK, _K), jnp.int32),
        pltpu.VMEM((_ROWS_PER_W,), jnp.int32),
        pltpu.VMEM((_ROWS_PER_W,), jnp.float32),
        pltpu.VMEM((_VOCAB // _NS,), jnp.float32),
        pltpu.VMEM((_NB, _K, _VOCAB), jnp.float32),
        pltpu.VMEM((_L,), jnp.float32),
        pltpu.VMEM_SHARED((_VOCAB,), jnp.float32),
        pltpu.VMEM_SHARED((_NS, 2, _K, _VOCAB), jnp.float32),
        pltpu.SemaphoreType.DMA((_NB,)),
        pltpu.SemaphoreType.DMA((_NB,)),
        pltpu.SemaphoreType.DMA((2,)),
        pltpu.SemaphoreType.DMA,
    ],
)
def _sc_gather(W_hbm, idx_hbm, idx2_hbm, tgt_hbm, out_hbm, part_hbm, hist_hbm,
               idx_v, idx2_v, tgt_v, ones_v, zsrc_v, rows_v, acc_v, hist_sh,
               sp_sh, gsems, csems, wsems, hsem):
    sid = lax.axis_index("s")
    cid = lax.axis_index("c")
    wid = sid * _NC + cid
    base = wid * _ROWS_PER_W
    pltpu.sync_copy(idx_hbm.at[pl.ds(base, _ROWS_PER_W)], idx_v)
    pltpu.sync_copy(idx2_hbm.at[pl.ds(wid * _NCHUNK, _NCHUNK)], idx2_v)
    pltpu.sync_copy(tgt_hbm.at[pl.ds(base, _ROWS_PER_W)], tgt_v)

    lanes = lax.iota(jnp.int32, _L)

    def _gather(c, b):
        return pltpu.async_copy(
            W_hbm.at[idx2_v.at[c]], rows_v.at[b], gsems.at[b])

    def _gather_wait(c, b):
        pltpu.make_async_copy(
            W_hbm.at[idx2_v.at[c]], rows_v.at[b],
            gsems.at[b]).wait()

    def _xb(b, s):
        return pltpu.async_copy(rows_v.at[b], sp_sh.at[sid, s], csems.at[b])

    def _xb_wait(b, s):
        pltpu.make_async_copy(rows_v.at[b], sp_sh.at[sid, s],
                              csems.at[b]).wait()

    def _wb(c, s):
        return pltpu.async_copy(
            sp_sh.at[sid, s], out_hbm.at[pl.ds(base + c * _K, _K)],
            wsems.at[s])

    def _wb_wait(c, s):
        pltpu.make_async_copy(
            sp_sh.at[sid, s], out_hbm.at[pl.ds(base + c * _K, _K)],
            wsems.at[s]).wait()

    _gather(0, 0)
    _gather(1, 1)

    _ZS = _VOCAB // _NS

    def obody(j, _):
        ones_v[pl.ds(j * _L, _L)] = jnp.ones((_L,), jnp.float32)
        return 0

    lax.fori_loop(0, _ROWS_PER_W // _L, obody, 0)

    def zb(j, _):
        zsrc_v[pl.ds(j * _L, _L)] = jnp.zeros((_L,), jnp.float32)
        return 0

    lax.fori_loop(0, _ZS // _L, zb, 0)
    pltpu.sync_copy(zsrc_v.at[pl.ds(0, _ZS)], hist_sh.at[pl.ds(sid * _ZS, _ZS)])
    plsc.subcore_barrier()
    cp_h = pltpu.async_copy(ones_v, hist_sh.at[idx_v], hsem, add=True)

    def _chunk_tvals(b, t16, accv):
        for j in range(_K):
            t_j = t16[b * _K + j]
            cbase = (t_j // _L) * _L
            sl = rows_v[b, j, pl.ds(cbase, _L)]
            accv = accv + jnp.where(lanes == (t_j % _L), sl, 0.0)
        return accv

    def body(g, acct):
        t16 = tgt_v[pl.ds(g * _L, _L)]
        for b in range(_NB):
            c = g * _NB + b
            s = b & 1
            _gather_wait(c, b)
            if b < 2:
                @pl.when(g > 0)
                def _():
                    _wb_wait(c - 2, s)
            else:
                _wb_wait(c - 2, s)
            _xb(b, s)
            acct = _chunk_tvals(b, t16, acct)
            _xb_wait(b, s)
            _wb(c, s)
            b2 = (b + 2) % _NB
            if b < 2:
                _gather(c + 2, b2)
            else:
                @pl.when(g < _NG - 1)
                def _():
                    _gather(c + 2, b2)
        return acct

    acct = lax.fori_loop(0, _NG, body, jnp.zeros((_L,), jnp.float32))
    _wb_wait(_NCHUNK - 2, 0)
    _wb_wait(_NCHUNK - 1, 1)

    acc_v[...] = acct
    pltpu.sync_copy(acc_v, part_hbm.at[wid])

    cp_h.wait()
    plsc.subcore_barrier()

    @pl.when(sid == 0)
    def _():
        pltpu.sync_copy(hist_sh, hist_hbm.at[cid])


def kernel(idx, targets, W):
    idx_flat = idx.reshape(_N).astype(jnp.int32)
    tgt_flat = targets.reshape(_N).astype(jnp.int32)
    lse2 = _compute_lse(W)
    logits_flat, tpart, hist = _sc_gather(
        W, idx_flat, idx_flat.reshape(_N // _K, _K), tgt_flat)
    loss = _combine(hist, lse2, tpart)[0, 0]
    return (logits_flat, loss)


# speedup vs baseline: 1.0558x; 1.0199x over previous
import functools

import jax
import jax.numpy as jnp
from jax import lax
from jax.experimental import pallas as pl
from jax.experimental.pallas import tpu as pltpu
from jax.experimental.pallas import tpu_sc as plsc

_VOCAB = 4096
_N = 8 * 2048
_NC, _NS, _L = 2, 16, 16
_NW = _NC * _NS
_ROWS_PER_W = _N // _NW
_K = 4
_NB = 4
_NCHUNK = _ROWS_PER_W // _K
_NG = _NCHUNK // _NB
_TPAD = _ROWS_PER_W + _L


def _lse_body(w_ref, out_ref):
    x = w_ref[...]
    m = jnp.max(x, axis=1, keepdims=True)
    s = jnp.sum(jnp.exp(x - m), axis=1, keepdims=True)
    out_ref[...] = m + jnp.log(s)


_R = 256


def _compute_lse(W):
    return pl.pallas_call(
        _lse_body,
        grid=(_VOCAB // _R,),
        in_specs=[pl.BlockSpec((_R, _VOCAB), lambda i: (i, 0))],
        out_specs=pl.BlockSpec((_R, 1), lambda i: (i, 0)),
        out_shape=jax.ShapeDtypeStruct((_VOCAB, 1), jnp.float32),
    )(W)


def _combine_body(hist_ref, lse_ref, tpart_ref, out_ref):
    counts = jnp.sum(hist_ref[...].astype(jnp.float32), axis=0,
                     keepdims=True)
    s_lse = jnp.dot(counts, lse_ref[...],
                    preferred_element_type=jnp.float32)
    s_tv = jnp.sum(tpart_ref[...])
    out_ref[...] = (s_lse - s_tv) / _N


def _combine(hist, lse2, tpart):
    return pl.pallas_call(
        _combine_body,
        out_shape=jax.ShapeDtypeStruct((1, 1), jnp.float32),
    )(hist, lse2, tpart)


@functools.partial(
    pl.kernel,
    mesh=plsc.VectorSubcoreMesh(core_axis_name="c", subcore_axis_name="s"),
    out_type=[
        jax.ShapeDtypeStruct((_N, _VOCAB), jnp.float32),
        jax.ShapeDtypeStruct((_NW, _L), jnp.float32),
        jax.ShapeDtypeStruct((_NC, _VOCAB), jnp.float32),
    ],
    scratch_types=[
        pltpu.VMEM((_ROWS_PER_W,), jnp.int32),
        pltpu.VMEM((_NCHUNK, _K), jnp.int32),
        pltpu.VMEM((_ROWS_PER_W,), jnp.int32),
        pltpu.VMEM((_ROWS_PER_W,), jnp.float32),
        pltpu.VMEM((_VOCAB // _NS,), jnp.float32),
        pltpu.VMEM((_NB, _K, _VOCAB), jnp.float32),
        pltpu.VMEM((_L,), jnp.float32),
        pltpu.VMEM_SHARED((_VOCAB,), jnp.float32),
        pltpu.VMEM_SHARED((_NS, 2, _K, _VOCAB), jnp.float32),
        pltpu.SemaphoreType.DMA((_NB,)),
        pltpu.SemaphoreType.DMA((_NB,)),
        pltpu.SemaphoreType.DMA((2,)),
        pltpu.SemaphoreType.DMA,
    ],
)
def _sc_gather(W_hbm, idx_hbm, idx2_hbm, tgt_hbm, out_hbm, part_hbm, hist_hbm,
               idx_v, idx2_v, tgt_v, ones_v, zsrc_v, rows_v, acc_v, hist_sh,
               sp_sh, gsems, csems, wsems, hsem):
    sid = lax.axis_index("s")
    cid = lax.axis_index("c")
    wid = sid * _NC + cid
    base = wid * _ROWS_PER_W
    pltpu.sync_copy(idx_hbm.at[pl.ds(base, _ROWS_PER_W)], idx_v)
    pltpu.sync_copy(idx2_hbm.at[pl.ds(wid * _NCHUNK, _NCHUNK)], idx2_v)
    pltpu.sync_copy(tgt_hbm.at[pl.ds(base, _ROWS_PER_W)], tgt_v)

    lanes = lax.iota(jnp.int32, _L)

    def _gather(c, b):
        return pltpu.async_copy(
            W_hbm.at[idx2_v.at[c]], rows_v.at[b], gsems.at[b])

    def _gather_wait(c, b):
        pltpu.make_async_copy(
            W_hbm.at[idx2_v.at[c]], rows_v.at[b],
            gsems.at[b]).wait()

    def _xb(b, s):
        return pltpu.async_copy(rows_v.at[b], sp_sh.at[sid, s], csems.at[b])

    def _xb_wait(b, s):
        pltpu.make_async_copy(rows_v.at[b], sp_sh.at[sid, s],
                              csems.at[b]).wait()

    def _wb(c, s):
        return pltpu.async_copy(
            sp_sh.at[sid, s], out_hbm.at[pl.ds(base + c * _K, _K)],
            wsems.at[s])

    def _wb_wait(c, s):
        pltpu.make_async_copy(
            sp_sh.at[sid, s], out_hbm.at[pl.ds(base + c * _K, _K)],
            wsems.at[s]).wait()

    _gather(0, 0)
    _gather(1, 1)

    _ZS = _VOCAB // _NS

    def obody(j, _):
        ones_v[pl.ds(j * _L, _L)] = jnp.ones((_L,), jnp.float32)
        return 0

    lax.fori_loop(0, _ROWS_PER_W // _L, obody, 0)

    def zb(j, _):
        zsrc_v[pl.ds(j * _L, _L)] = jnp.zeros((_L,), jnp.float32)
        return 0

    lax.fori_loop(0, _ZS // _L, zb, 0)
    pltpu.sync_copy(zsrc_v.at[pl.ds(0, _ZS)], hist_sh.at[pl.ds(sid * _ZS, _ZS)])
    plsc.subcore_barrier()
    cp_h = pltpu.async_copy(ones_v, hist_sh.at[idx_v], hsem, add=True)

    def _chunk_tvals(b, t16, accv):
        for j in range(_K):
            t_j = t16[b * _K + j]
            cbase = (t_j // _L) * _L
            sl = rows_v[b, j, pl.ds(cbase, _L)]
            accv = accv + jnp.where(lanes == (t_j % _L), sl, 0.0)
        return accv

    def body(g, acct):
        t16 = tgt_v[pl.ds(g * _L, _L)]
        for b in range(_NB):
            c = g * _NB + b
            s = b & 1
            _gather_wait(c, b)
            if b < 2:
                @pl.when(g > 0)
                def _():
                    _wb_wait(c - 2, s)
            else:
                _wb_wait(c - 2, s)
            _xb(b, s)
            b2 = (b + 2) % _NB
            if b < 2:
                _gather(c + 2, b2)
            else:
                @pl.when(g < _NG - 1)
                def _():
                    _gather(c + 2, b2)
            acct = _chunk_tvals(b, t16, acct)
            _xb_wait(b, s)
            _wb(c, s)
        return acct

    acct = lax.fori_loop(0, _NG, body, jnp.zeros((_L,), jnp.float32))
    _wb_wait(_NCHUNK - 2, 0)
    _wb_wait(_NCHUNK - 1, 1)

    acc_v[...] = acct
    pltpu.sync_copy(acc_v, part_hbm.at[wid])

    cp_h.wait()
    plsc.subcore_barrier()

    @pl.when(sid == 0)
    def _():
        pltpu.sync_copy(hist_sh, hist_hbm.at[cid])


def kernel(idx, targets, W):
    idx_flat = idx.reshape(_N).astype(jnp.int32)
    tgt_flat = targets.reshape(_N).astype(jnp.int32)
    lse2 = _compute_lse(W)
    logits_flat, tpart, hist = _sc_gather(
        W, idx_flat, idx_flat.reshape(_N // _K, _K), tgt_flat)
    loss = _combine(hist, lse2, tpart)[0, 0]
    return (logits_flat, loss)
